# Initial kernel scaffold; baseline (speedup 1.0000x reference)
import jax
import jax.numpy as jnp
from jax.experimental import pallas as pl


def kernel(soft_label, pixel_weights):
    raise NotImplementedError("write your pallas kernel here")


# speedup vs baseline: 404.5086x; 404.5086x over previous
import functools

import jax
import jax.numpy as jnp
from jax.experimental import pallas as pl
from jax.experimental.pallas import tpu as pltpu

_NUM_CLASSES = 19
_BH = 128


def _weighter_body(pw_ref, x_ref, o_ref):
    x = x_ref[0]
    m = x[0]
    w = jnp.full_like(m, pw_ref[0])
    for c in range(1, _NUM_CLASSES):
        v = x[c]
        gt = v > m
        m = jnp.where(gt, v, m)
        w = jnp.where(gt, pw_ref[c], w)
    o_ref[0] = w


@jax.jit
def kernel(soft_label, pixel_weights):
    b, nc, h, wdim = soft_label.shape
    grid = (b, h // _BH)
    return pl.pallas_call(
        _weighter_body,
        grid=grid,
        in_specs=[
            pl.BlockSpec(memory_space=pltpu.SMEM),
            pl.BlockSpec((1, nc, _BH, wdim), lambda i, j: (i, 0, j, 0)),
        ],
        out_specs=pl.BlockSpec((1, _BH, wdim), lambda i, j: (i, j, 0)),
        out_shape=jax.ShapeDtypeStruct((b, h, wdim), jnp.float32),
    )(pixel_weights, soft_label)
